# baseline (speedup 1.0000x reference)
import jax
import jax.numpy as jnp
import numpy as np
from jax.experimental import pallas as pl
from jax.experimental.pallas import tpu as pltpu

D = 768
PS = 16
IMG = 224
NPATCH = (IMG // PS) ** 2
PD = PS * PS
NE = 8
NH = 8
DH = D // NH
HID = 256
B = 32
T = B * NPATCH
HP = jax.lax.Precision.HIGHEST
SCALE = 1.0 / np.sqrt(DH)

_STD = (((1,), (0,)), ((), ()))
_TR = (((1,), (1,)), ((), ()))

MA = 784
MO = 1568
ME = 1568
MH = 784
AC = 2


def _dot(a, b):
    return jnp.dot(a, b, preferred_element_type=jnp.float32, precision=HP)


def _dotg(a, b, dims):
    return jax.lax.dot_general(a, b, dims,
                               preferred_element_type=jnp.float32,
                               precision=HP)


def _split(a):
    ah = a.astype(jnp.bfloat16)
    return ah, (a - ah.astype(jnp.float32)).astype(jnp.bfloat16)


def _dotf(a, b):
    ah, al = _split(a)
    bh, bl = _split(b)
    d = lambda u, v: jax.lax.dot_general(
        u, v, _STD, preferred_element_type=jnp.float32)
    return d(ah, bh) + d(ah, bl) + d(al, bh)


def _gate_weights(logits):
    m = jnp.max(logits, axis=-1, keepdims=True)
    e = jnp.exp(logits - m)
    p = e / jnp.sum(e, axis=-1, keepdims=True)
    idx = jax.lax.broadcasted_iota(jnp.int32, p.shape, 1)
    p1 = jnp.max(p, axis=-1, keepdims=True)
    i1 = jnp.min(jnp.where(p == p1, idx, NE), axis=-1, keepdims=True)
    pm = jnp.where(idx == i1, -jnp.inf, p)
    p2 = jnp.max(pm, axis=-1, keepdims=True)
    i2 = jnp.min(jnp.where(pm == p2, idx, NE), axis=-1, keepdims=True)
    return jnp.where((idx == i1) | (idx == i2), p, 0.0) / (p1 + p2)


def _qkv1_body(x_ref, posb_ref, pwt_ref, inwt_ref, inb_ref, qwt_ref, qb_ref,
               out_ref):
    x2 = _dot(x_ref[...], pwt_ref[...]) + posb_ref[...]
    xi = _dot(x2, inwt_ref[...]) + inb_ref[...]
    out_ref[...] = _dot(xi, qwt_ref[...]) + qb_ref[...]


def _qkv2_body(x_ref, inwt_ref, inb_ref, qwt_ref, qb_ref, out_ref):
    xi = _dot(x_ref[...], inwt_ref[...]) + inb_ref[...]
    out_ref[...] = _dot(xi, qwt_ref[...]) + qb_ref[...]


def _attn_body(qkv_ref, out_ref):
    n = AC * NH * B
    qs, ks, vs = [], [], []
    for cc in range(AC):
        x3 = qkv_ref[:, 0, cc, :]
        for h in range(NH):
            qs.append(x3[:, h * DH:(h + 1) * DH])
            ks.append(x3[:, D + h * DH:D + (h + 1) * DH])
            vs.append(x3[:, 2 * D + h * DH:2 * D + (h + 1) * DH])
    q8 = jnp.concatenate(qs, 0)
    k8 = jnp.concatenate(ks, 0)
    v8 = jnp.concatenate(vs, 0)
    s = _dotg(q8, k8, _TR) * SCALE
    blk = (jax.lax.broadcasted_iota(jnp.int32, (n, n), 0) // B ==
           jax.lax.broadcasted_iota(jnp.int32, (n, n), 1) // B)
    s = jnp.where(blk, s, -1e30)
    m = jnp.max(s, axis=-1, keepdims=True)
    e = jnp.exp(s - m)
    pa = e / jnp.sum(e, axis=-1, keepdims=True)
    o8 = _dot(pa, v8)
    pieces = []
    for cc in range(AC):
        g = cc * NH
        rows = jnp.concatenate(
            [o8[(g + h) * B:(g + h + 1) * B, :] for h in range(NH)], 1)
        pieces.append(rows.reshape(B, 1, 1, D))
    out_ref[...] = jnp.concatenate(pieces, 2)


def _owg_body(ao_ref, owt_ref, ob_ref, gwt_ref, gb_ref, xo_ref, wi_ref):
    xo = _dot(ao_ref[...], owt_ref[...]) + ob_ref[...]
    xo_ref[...] = xo
    wi_ref[...] = _gate_weights(_dot(xo, gwt_ref[...]) + gb_ref[...])


def _make_exp_body(fast):
    dd = _dotf if fast else _dot

    def body(xo_ref, wi_ref, w1t_ref, b1_ref, w2t_ref, b2_ref, out_ref):
        xo = xo_ref[...]
        wi = wi_ref[...]

        def exp_step(i, acc):
            w1 = w1t_ref[pl.ds(i, 1)].reshape(D, HID)
            h = jnp.maximum(dd(xo, w1) + b1_ref[pl.ds(i, 1)].reshape(1, HID),
                            0.0)
            w2 = w2t_ref[pl.ds(i, 1)].reshape(HID, D)
            eo = dd(h, w2) + b2_ref[pl.ds(i, 1)].reshape(1, D)
            eidx = jax.lax.broadcasted_iota(jnp.int32, wi.shape, 1)
            ws = jnp.sum(jnp.where(eidx == i, wi, 0.0), axis=1, keepdims=True)
            return acc + eo * ws

        out_ref[...] = jax.lax.fori_loop(
            0, NE, exp_step, jnp.zeros((xo.shape[0], D), jnp.float32))

    return body


_exp_body_hi = _make_exp_body(False)
_exp_body_fast = _make_exp_body(True)


def _make_heads_body(fast_fv):
    dfv = _dotf if fast_fv else _dot

    def body(x_ref, vwt_ref, vb_ref, cwt_ref, cb_ref, fv_ref, cls_ref):
        x = x_ref[...]
        fv_ref[...] = dfv(x, vwt_ref[...]) + vb_ref[...]
        cls_ref[...] = _dotf(x, cwt_ref[...]) + cb_ref[...]

    return body


_heads_body_hi = _make_heads_body(False)
_heads_body_fast = _make_heads_body(True)


def _c2(shp):
    return pl.BlockSpec(shp, lambda i: (0, 0))


def _c3(shp):
    return pl.BlockSpec(shp, lambda i: (0, 0, 0))


def _rows(bm, n):
    return pl.BlockSpec((bm, n), lambda i: (i, 0))


def _f32(shape):
    return jax.ShapeDtypeStruct(shape, jnp.float32)


def _qkv_layer(xin, posb, pwt, mp):
    if posb is not None:
        return pl.pallas_call(
            _qkv1_body,
            grid=(T // MA,),
            in_specs=[_rows(MA, PD), _c2((MA, D)), _c2((PD, D)),
                      _c2((D, D)), _c2((1, D)),
                      _c2((D, 3 * D)), _c2((1, 3 * D))],
            out_specs=_rows(MA, 3 * D),
            out_shape=_f32((T, 3 * D)),
        )(xin, posb, pwt, mp["inW"].T, mp["inb"].reshape(1, D),
          mp["qkvW"].T, mp["qkvb"].reshape(1, 3 * D))
    return pl.pallas_call(
        _qkv2_body,
        grid=(T // MA,),
        in_specs=[_rows(MA, D), _c2((D, D)), _c2((1, D)),
                  _c2((D, 3 * D)), _c2((1, 3 * D))],
        out_specs=_rows(MA, 3 * D),
        out_shape=_f32((T, 3 * D)),
    )(xin, mp["inW"].T, mp["inb"].reshape(1, D),
      mp["qkvW"].T, mp["qkvb"].reshape(1, 3 * D))


def _attn_layer(qkv):
    qkv4 = qkv.reshape(B, NPATCH // AC, AC, 3 * D)
    out = pl.pallas_call(
        _attn_body,
        grid=(NPATCH // AC,),
        in_specs=[pl.BlockSpec((B, 1, AC, 3 * D), lambda i: (0, i, 0, 0))],
        out_specs=pl.BlockSpec((B, 1, AC, D), lambda i: (0, i, 0, 0)),
        out_shape=_f32((B, NPATCH // AC, AC, D)),
    )(qkv4)
    return out.reshape(T, D)


def _owg_layer(ao, mp):
    return pl.pallas_call(
        _owg_body,
        grid=(T // MO,),
        in_specs=[_rows(MO, D), _c2((D, D)), _c2((1, D)),
                  _c2((D, NE)), _c2((1, NE))],
        out_specs=[_rows(MO, D), _rows(MO, NE)],
        out_shape=[_f32((T, D)), _f32((T, NE))],
    )(ao, mp["oW"].T, mp["ob"].reshape(1, D),
      mp["gW"].T, mp["gb"].reshape(1, NE))


def _exp_layer(xo, wi, mp, fast):
    return pl.pallas_call(
        _exp_body_fast if fast else _exp_body_hi,
        grid=(T // ME,),
        in_specs=[_rows(ME, D), _rows(ME, NE),
                  _c3((NE, D, HID)), _c3((NE, 1, HID)),
                  _c3((NE, HID, D)), _c3((NE, 1, D))],
        out_specs=_rows(ME, D),
        out_shape=_f32((T, D)),
    )(xo, wi, mp["W1"].transpose(0, 2, 1), mp["b1"].reshape(NE, 1, HID),
      mp["W2"].transpose(0, 2, 1), mp["b2"].reshape(NE, 1, D))


def _heads_layer(xf, vWt, vb, cWt, cb, fast_fv):
    return pl.pallas_call(
        _heads_body_fast if fast_fv else _heads_body_hi,
        grid=(T // MH,),
        in_specs=[_rows(MH, D), _c2((D, D)), _c2((1, D)),
                  _c2((D, D)), _c2((1, D))],
        out_specs=[_rows(MH, D), _rows(MH, D)],
        out_shape=[_f32((T, D)), _f32((T, D))],
    )(xf, vWt, vb, cWt, cb)


def _moe_layer(xin, posb, pwt, mp, fast_tail):
    qkv = _qkv_layer(xin, posb, pwt, mp)
    ao = _attn_layer(qkv)
    xo, wi = _owg_layer(ao, mp)
    return _exp_layer(xo, wi, mp, fast_tail)


def kernel(x, params):
    n = IMG // PS
    xp = (x.reshape(B, n, PS, n, PS)
           .transpose(0, 1, 3, 2, 4)
           .reshape(T, PD))
    posn = params["pos"].reshape(NPATCH, D) + params["pb"].reshape(1, D)
    posb = jnp.tile(posn, (MA // NPATCH, 1))
    vWt = params["vW"].T
    vb = params["vb"].reshape(1, D)
    cWt = params["cW"].T
    cb = params["cb"].reshape(1, D)

    first = _moe_layer(xp, posb, params["pW"].T, params["moe1"], False)
    fv1, cls1 = _heads_layer(first, vWt, vb, cWt, cb, False)
    second = _moe_layer(fv1, None, None, params["moe2"], True)
    fv2, cls2 = _heads_layer(second, vWt, vb, cWt, cb, True)

    sh = (B, NPATCH, D)
    return (fv1.reshape(sh), fv2.reshape(sh),
            cls1.reshape(sh), cls2.reshape(sh))


# speedup vs baseline: 1.1367x; 1.1367x over previous
import functools

import jax
import jax.numpy as jnp
import numpy as np
from jax.experimental import pallas as pl
from jax.experimental.pallas import tpu as pltpu

D = 768
PS = 16
IMG = 224
NPATCH = (IMG // PS) ** 2
PD = PS * PS
NE = 8
NH = 8
DH = D // NH
HID = 256
B = 32
T = B * NPATCH
P = 14
HP = jax.lax.Precision.HIGHEST

SCALE = 1.0 / np.sqrt(DH)


_STD = (((1,), (0,)), ((), ()))
_TR = (((1,), (1,)), ((), ()))


def _dot(a, b):
    return jnp.dot(a, b, preferred_element_type=jnp.float32, precision=HP)


def _split(a):
    ah = a.astype(jnp.bfloat16)
    return ah, (a - ah.astype(jnp.float32)).astype(jnp.bfloat16)


def _dotf(a, b):
    ah, al = _split(a)
    bh, bl = _split(b)
    d = lambda u, v: jax.lax.dot_general(
        u, v, _STD, preferred_element_type=jnp.float32)
    return d(ah, bh) + d(ah, bl) + d(al, bh)


AC = 2


def _attn_group(x3s):
    qs, ks, vs = [], [], []
    for x3 in x3s:
        for h in range(NH):
            qs.append(x3[:, h * DH:(h + 1) * DH])
            ks.append(x3[:, D + h * DH:D + (h + 1) * DH])
            vs.append(x3[:, 2 * D + h * DH:2 * D + (h + 1) * DH])
    q8 = jnp.concatenate(qs, 0)
    k8 = jnp.concatenate(ks, 0)
    v8 = jnp.concatenate(vs, 0)
    s = jax.lax.dot_general(q8, k8, _TR, preferred_element_type=jnp.float32,
                            precision=HP) * SCALE
    n = AC * NH * B
    blk = (jax.lax.broadcasted_iota(jnp.int32, (n, n), 0) // B ==
           jax.lax.broadcasted_iota(jnp.int32, (n, n), 1) // B)
    s = jnp.where(blk, s, -1e30)
    m = jnp.max(s, axis=-1, keepdims=True)
    e = jnp.exp(s - m)
    pa = e / jnp.sum(e, axis=-1, keepdims=True)
    o8 = _dot(pa, v8)
    outs = []
    for cc in range(AC):
        g = cc * NH
        rows = jnp.concatenate(
            [o8[(g + h) * B:(g + h + 1) * B, :] for h in range(NH)], 1)
        outs.append(rows.reshape(B, 1, D))
    return jnp.concatenate(outs, 1)


def _gate_weights(logits):
    m = jnp.max(logits, axis=-1, keepdims=True)
    e = jnp.exp(logits - m)
    p = e / jnp.sum(e, axis=-1, keepdims=True)
    idx = jax.lax.broadcasted_iota(jnp.int32, p.shape, 1)
    p1 = jnp.max(p, axis=-1, keepdims=True)
    i1 = jnp.min(jnp.where(p == p1, idx, NE), axis=-1, keepdims=True)
    pm = jnp.where(idx == i1, -jnp.inf, p)
    p2 = jnp.max(pm, axis=-1, keepdims=True)
    i2 = jnp.min(jnp.where(pm == p2, idx, NE), axis=-1, keepdims=True)
    return jnp.where((idx == i1) | (idx == i2), p, 0.0) / (p1 + p2)


def _moe_body(x2, refs, qkv_s, o_s, fast_tail):
    (inwt, inb, qkvwt, qkvb, owt, ob, gwt, gb, w1t, b1, w2t, b2,
     vwt, vb, cwt, cb) = refs
    rows = x2.shape[0]
    xi = _dot(x2, inwt[...]) + inb[...]
    qkv = _dot(xi, qkvwt[...]) + qkvb[...]
    qkv_s[...] = qkv.reshape(B, P, 3 * D)

    def attn_step(g_, _):
        x3s = [qkv_s[:, pl.ds(g_ * AC + cc, 1), :].reshape(B, 3 * D)
               for cc in range(AC)]
        og = _attn_group(x3s)
        for cc in range(AC):
            o_s[:, pl.ds(g_ * AC + cc, 1), :] = og[:, cc:cc + 1, :]
        return 0

    jax.lax.fori_loop(0, P // AC, attn_step, 0)
    xo = _dot(o_s[...].reshape(rows, D), owt[...]) + ob[...]
    wi = _gate_weights(_dot(xo, gwt[...]) + gb[...])
    edot = _dotf if fast_tail else _dot

    def exp_step(i, acc):
        w1 = w1t[pl.ds(i, 1)].reshape(D, HID)
        h = jnp.maximum(edot(xo, w1) + b1[pl.ds(i, 1)].reshape(1, HID), 0.0)
        w2 = w2t[pl.ds(i, 1)].reshape(HID, D)
        eo = edot(h, w2) + b2[pl.ds(i, 1)].reshape(1, D)
        eidx = jax.lax.broadcasted_iota(jnp.int32, wi.shape, 1)
        wsel = jnp.sum(jnp.where(eidx == i, wi, 0.0), axis=1, keepdims=True)
        return acc + eo * wsel

    acc = jax.lax.fori_loop(0, NE, exp_step, jnp.zeros((rows, D), jnp.float32))
    fv = (edot(acc, vwt[...]) + vb[...])
    cls = _dotf(acc, cwt[...]) + cb[...]
    return fv.reshape(B, P, D), cls.reshape(B, P, D)


def _k1_body(xp_ref, pos_ref, pwt_ref, *rest):
    refs = rest[:16]
    fv_ref, cls_ref, qkv_s, o_s = rest[16:]
    x = xp_ref[...].reshape(B * P, PD)
    x2 = _dot(x, pwt_ref[...])
    x2 = (x2.reshape(B, P, D) + pos_ref[0][None]).reshape(B * P, D)
    fv, cls = _moe_body(x2, refs, qkv_s, o_s, False)
    fv_ref[...] = fv.reshape(B, 1, P, D)
    cls_ref[...] = cls.reshape(B, 1, P, D)


def _k2_body(xin_ref, *rest):
    refs = rest[:16]
    fv_ref, cls_ref, qkv_s, o_s = rest[16:]
    x2 = xin_ref[...].reshape(B * P, D)
    fv, cls = _moe_body(x2, refs, qkv_s, o_s, True)
    fv_ref[...] = fv.reshape(B, 1, P, D)
    cls_ref[...] = cls.reshape(B, 1, P, D)


def _moe_args(mp, vWt, vb, cWt, cb):
    return (
        mp["inW"].T, mp["inb"].reshape(1, D),
        mp["qkvW"].T, mp["qkvb"].reshape(1, 3 * D),
        mp["oW"].T, mp["ob"].reshape(1, D),
        mp["gW"].T, mp["gb"].reshape(1, NE),
        mp["W1"].transpose(0, 2, 1), mp["b1"].reshape(NE, 1, HID),
        mp["W2"].transpose(0, 2, 1), mp["b2"].reshape(NE, 1, D),
        vWt, vb, cWt, cb,
    )


def _moe_specs():
    c2 = lambda shp: pl.BlockSpec(shp, lambda i: (0, 0))
    c3 = lambda shp: pl.BlockSpec(shp, lambda i: (0, 0, 0))
    return [
        c2((D, D)), c2((1, D)),
        c2((D, 3 * D)), c2((1, 3 * D)),
        c2((D, D)), c2((1, D)),
        c2((D, NE)), c2((1, NE)),
        c3((NE, D, HID)), c3((NE, 1, HID)),
        c3((NE, HID, D)), c3((NE, 1, D)),
        c2((D, D)), c2((1, D)), c2((D, D)), c2((1, D)),
    ]


NG = NPATCH // P


def _blk(last):
    return pl.BlockSpec((B, 1, P, last), lambda i: (0, i, 0, 0))


def kernel(x, params):
    n = IMG // PS
    xp4 = (x.reshape(B, n, PS, n, PS)
            .transpose(0, 1, 3, 2, 4)
            .reshape(B, NG, P, PD))
    pos = (params["pos"].reshape(NPATCH, D) +
           params["pb"].reshape(1, D)).reshape(NG, P, D)
    vWt = params["vW"].T
    vb = params["vb"].reshape(1, D)
    cWt = params["cW"].T
    cb = params["cb"].reshape(1, D)
    out_sh = jax.ShapeDtypeStruct((B, NG, P, D), jnp.float32)

    fv1, cls1 = pl.pallas_call(
        _k1_body,
        grid=(NG,),
        in_specs=[_blk(PD),
                  pl.BlockSpec((1, P, D), lambda i: (i, 0, 0)),
                  pl.BlockSpec((PD, D), lambda i: (0, 0))] + _moe_specs(),
        out_specs=[_blk(D), _blk(D)],
        out_shape=[out_sh, out_sh],
        scratch_shapes=[pltpu.VMEM((B, P, 3 * D), jnp.float32),
                        pltpu.VMEM((B, P, D), jnp.float32)],
    )(xp4, pos, params["pW"].T, *_moe_args(params["moe1"], vWt, vb, cWt, cb))

    fv2, cls2 = pl.pallas_call(
        _k2_body,
        grid=(NG,),
        in_specs=[_blk(D)] + _moe_specs(),
        out_specs=[_blk(D), _blk(D)],
        out_shape=[out_sh, out_sh],
        scratch_shapes=[pltpu.VMEM((B, P, 3 * D), jnp.float32),
                        pltpu.VMEM((B, P, D), jnp.float32)],
    )(fv1, *_moe_args(params["moe2"], vWt, vb, cWt, cb))

    sh = (B, NPATCH, D)
    return (fv1.reshape(sh), fv2.reshape(sh),
            cls1.reshape(sh), cls2.reshape(sh))


# speedup vs baseline: 1.4959x; 1.3160x over previous
import functools

import jax
import jax.numpy as jnp
import numpy as np
from jax.experimental import pallas as pl
from jax.experimental.pallas import tpu as pltpu

D = 768
PS = 16
IMG = 224
NPATCH = (IMG // PS) ** 2
PD = PS * PS
NE = 8
NH = 8
DH = D // NH
HID = 256
B = 32
T = B * NPATCH
P = 14
HP = jax.lax.Precision.HIGHEST

SCALE = 1.0 / np.sqrt(DH)


_STD = (((1,), (0,)), ((), ()))
_TR = (((1,), (1,)), ((), ()))


def _dot(a, b):
    return jnp.dot(a, b, preferred_element_type=jnp.float32, precision=HP)


def _split(a):
    ah = a.astype(jnp.bfloat16)
    return ah, (a - ah.astype(jnp.float32)).astype(jnp.bfloat16)


def _dotf(a, b):
    ah, al = _split(a)
    bh, bl = _split(b)
    d = lambda u, v: jax.lax.dot_general(
        u, v, _STD, preferred_element_type=jnp.float32)
    return d(ah, bh) + d(ah, bl) + d(al, bh)


def _attn_patch(q, k, v):
    q8 = jnp.concatenate([q[:, h * DH:(h + 1) * DH] for h in range(NH)], 0)
    k8 = jnp.concatenate([k[:, h * DH:(h + 1) * DH] for h in range(NH)], 0)
    v8 = jnp.concatenate([v[:, h * DH:(h + 1) * DH] for h in range(NH)], 0)
    s = jax.lax.dot_general(q8, k8, _TR, preferred_element_type=jnp.float32,
                            precision=HP) * SCALE
    n = NH * B
    blk = (jax.lax.broadcasted_iota(jnp.int32, (n, n), 0) // B ==
           jax.lax.broadcasted_iota(jnp.int32, (n, n), 1) // B)
    s = jnp.where(blk, s, -1e30)
    m = jnp.max(s, axis=-1, keepdims=True)
    e = jnp.exp(s - m)
    pa = e / jnp.sum(e, axis=-1, keepdims=True)
    o8 = _dot(pa, v8)
    return jnp.concatenate([o8[h * B:(h + 1) * B, :] for h in range(NH)], 1)


def _gate_weights(logits):
    m = jnp.max(logits, axis=-1, keepdims=True)
    e = jnp.exp(logits - m)
    p = e / jnp.sum(e, axis=-1, keepdims=True)
    idx = jax.lax.broadcasted_iota(jnp.int32, p.shape, 1)
    p1 = jnp.max(p, axis=-1, keepdims=True)
    i1 = jnp.min(jnp.where(p == p1, idx, NE), axis=-1, keepdims=True)
    pm = jnp.where(idx == i1, -jnp.inf, p)
    p2 = jnp.max(pm, axis=-1, keepdims=True)
    i2 = jnp.min(jnp.where(pm == p2, idx, NE), axis=-1, keepdims=True)
    return jnp.where((idx == i1) | (idx == i2), p, 0.0) / (p1 + p2)


def _moe_body(x2, refs, qkv_s, o_s, fast_tail):
    (inwt, inb, qkvwt, qkvb, owt, ob, gwt, gb, w1t, b1, w2t, b2,
     vwt, vb, cwt, cb) = refs
    rows = x2.shape[0]
    xi = _dot(x2, inwt[...]) + inb[...]
    qkv = _dot(xi, qkvwt[...]) + qkvb[...]
    qkv_s[...] = qkv.reshape(B, P, 3 * D)

    def attn_step(p_, _):
        xp3 = qkv_s[:, pl.ds(p_, 1), :].reshape(B, 3 * D)
        o = _attn_patch(xp3[:, :D], xp3[:, D:2 * D], xp3[:, 2 * D:])
        o_s[:, pl.ds(p_, 1), :] = o.reshape(B, 1, D)
        return 0

    jax.lax.fori_loop(0, P, attn_step, 0, unroll=2)
    xo = _dot(o_s[...].reshape(rows, D), owt[...]) + ob[...]
    wi = _gate_weights(_dot(xo, gwt[...]) + gb[...])
    edot = _dotf if fast_tail else _dot

    def exp_step(i, acc):
        w1 = w1t[pl.ds(i, 1)].reshape(D, HID)
        h = jnp.maximum(edot(xo, w1) + b1[pl.ds(i, 1)].reshape(1, HID), 0.0)
        w2 = w2t[pl.ds(i, 1)].reshape(HID, D)
        eo = edot(h, w2) + b2[pl.ds(i, 1)].reshape(1, D)
        eidx = jax.lax.broadcasted_iota(jnp.int32, wi.shape, 1)
        wsel = jnp.sum(jnp.where(eidx == i, wi, 0.0), axis=1, keepdims=True)
        return acc + eo * wsel

    acc = jax.lax.fori_loop(0, NE, exp_step,
                            jnp.zeros((rows, D), jnp.float32), unroll=2)
    fv = (edot(acc, vwt[...]) + vb[...])
    cls = _dotf(acc, cwt[...]) + cb[...]
    return fv.reshape(B, P, D), cls.reshape(B, P, D)


def _k1_body(xp_ref, pos_ref, pwt_ref, *rest):
    refs = rest[:16]
    fv_ref, cls_ref, qkv_s, o_s = rest[16:]
    x = xp_ref[...].reshape(B * P, PD)
    x2 = _dot(x, pwt_ref[...])
    x2 = (x2.reshape(B, P, D) + pos_ref[0][None]).reshape(B * P, D)
    fv, cls = _moe_body(x2, refs, qkv_s, o_s, False)
    fv_ref[...] = fv.reshape(B, 1, P, D)
    cls_ref[...] = cls.reshape(B, 1, P, D)


def _k2_body(xin_ref, *rest):
    refs = rest[:16]
    fv_ref, cls_ref, qkv_s, o_s = rest[16:]
    x2 = xin_ref[...].reshape(B * P, D)
    fv, cls = _moe_body(x2, refs, qkv_s, o_s, True)
    fv_ref[...] = fv.reshape(B, 1, P, D)
    cls_ref[...] = cls.reshape(B, 1, P, D)


def _moe_args(mp, vWt, vb, cWt, cb):
    return (
        mp["inW"].T, mp["inb"].reshape(1, D),
        mp["qkvW"].T, mp["qkvb"].reshape(1, 3 * D),
        mp["oW"].T, mp["ob"].reshape(1, D),
        mp["gW"].T, mp["gb"].reshape(1, NE),
        mp["W1"].transpose(0, 2, 1), mp["b1"].reshape(NE, 1, HID),
        mp["W2"].transpose(0, 2, 1), mp["b2"].reshape(NE, 1, D),
        vWt, vb, cWt, cb,
    )


def _moe_specs():
    c2 = lambda shp: pl.BlockSpec(shp, lambda i: (0, 0))
    c3 = lambda shp: pl.BlockSpec(shp, lambda i: (0, 0, 0))
    return [
        c2((D, D)), c2((1, D)),
        c2((D, 3 * D)), c2((1, 3 * D)),
        c2((D, D)), c2((1, D)),
        c2((D, NE)), c2((1, NE)),
        c3((NE, D, HID)), c3((NE, 1, HID)),
        c3((NE, HID, D)), c3((NE, 1, D)),
        c2((D, D)), c2((1, D)), c2((D, D)), c2((1, D)),
    ]


NG = NPATCH // P


def _blk(last):
    return pl.BlockSpec((B, 1, P, last), lambda i: (0, i, 0, 0))


def kernel(x, params):
    n = IMG // PS
    xp4 = (x.reshape(B, n, PS, n, PS)
            .transpose(0, 1, 3, 2, 4)
            .reshape(B, NG, P, PD))
    pos = (params["pos"].reshape(NPATCH, D) +
           params["pb"].reshape(1, D)).reshape(NG, P, D)
    vWt = params["vW"].T
    vb = params["vb"].reshape(1, D)
    cWt = params["cW"].T
    cb = params["cb"].reshape(1, D)
    out_sh = jax.ShapeDtypeStruct((B, NG, P, D), jnp.float32)

    fv1, cls1 = pl.pallas_call(
        _k1_body,
        grid=(NG,),
        in_specs=[_blk(PD),
                  pl.BlockSpec((1, P, D), lambda i: (i, 0, 0)),
                  pl.BlockSpec((PD, D), lambda i: (0, 0))] + _moe_specs(),
        out_specs=[_blk(D), _blk(D)],
        out_shape=[out_sh, out_sh],
        scratch_shapes=[pltpu.VMEM((B, P, 3 * D), jnp.float32),
                        pltpu.VMEM((B, P, D), jnp.float32)],
    )(xp4, pos, params["pW"].T, *_moe_args(params["moe1"], vWt, vb, cWt, cb))

    fv2, cls2 = pl.pallas_call(
        _k2_body,
        grid=(NG,),
        in_specs=[_blk(D)] + _moe_specs(),
        out_specs=[_blk(D), _blk(D)],
        out_shape=[out_sh, out_sh],
        scratch_shapes=[pltpu.VMEM((B, P, 3 * D), jnp.float32),
                        pltpu.VMEM((B, P, D), jnp.float32)],
    )(fv1, *_moe_args(params["moe2"], vWt, vb, cWt, cb))

    sh = (B, NPATCH, D)
    return (fv1.reshape(sh), fv2.reshape(sh),
            cls1.reshape(sh), cls2.reshape(sh))


# speedup vs baseline: 1.5150x; 1.0128x over previous
import functools

import jax
import jax.numpy as jnp
import numpy as np
from jax.experimental import pallas as pl
from jax.experimental.pallas import tpu as pltpu

D = 768
PS = 16
IMG = 224
NPATCH = (IMG // PS) ** 2
PD = PS * PS
NE = 8
NH = 8
DH = D // NH
HID = 256
B = 32
T = B * NPATCH
P = 14
HP = jax.lax.Precision.HIGHEST

SCALE = 1.0 / np.sqrt(DH)


_STD = (((1,), (0,)), ((), ()))
_TR = (((1,), (1,)), ((), ()))


def _dot(a, b):
    return jnp.dot(a, b, preferred_element_type=jnp.float32, precision=HP)


def _split(a):
    ah = a.astype(jnp.bfloat16)
    return ah, (a - ah.astype(jnp.float32)).astype(jnp.bfloat16)


def _dotf(a, b):
    ah, al = _split(a)
    bh, bl = _split(b)
    d = lambda u, v: jax.lax.dot_general(
        u, v, _STD, preferred_element_type=jnp.float32)
    return d(ah, bh) + d(ah, bl) + d(al, bh)


def _attn_patch(q, k, v):
    q8 = jnp.concatenate([q[:, h * DH:(h + 1) * DH] for h in range(NH)], 0)
    k8 = jnp.concatenate([k[:, h * DH:(h + 1) * DH] for h in range(NH)], 0)
    v8 = jnp.concatenate([v[:, h * DH:(h + 1) * DH] for h in range(NH)], 0)
    s = jax.lax.dot_general(q8, k8, _TR, preferred_element_type=jnp.float32,
                            precision=HP) * SCALE
    n = NH * B
    blk = (jax.lax.broadcasted_iota(jnp.int32, (n, n), 0) // B ==
           jax.lax.broadcasted_iota(jnp.int32, (n, n), 1) // B)
    s = jnp.where(blk, s, -1e30)
    m = jnp.max(s, axis=-1, keepdims=True)
    e = jnp.exp(s - m)
    pa = e / jnp.sum(e, axis=-1, keepdims=True)
    o8 = _dot(pa, v8)
    return jnp.concatenate([o8[h * B:(h + 1) * B, :] for h in range(NH)], 1)


def _gate_weights(logits):
    m = jnp.max(logits, axis=-1, keepdims=True)
    e = jnp.exp(logits - m)
    p = e / jnp.sum(e, axis=-1, keepdims=True)
    idx = jax.lax.broadcasted_iota(jnp.int32, p.shape, 1)
    p1 = jnp.max(p, axis=-1, keepdims=True)
    i1 = jnp.min(jnp.where(p == p1, idx, NE), axis=-1, keepdims=True)
    pm = jnp.where(idx == i1, -jnp.inf, p)
    p2 = jnp.max(pm, axis=-1, keepdims=True)
    i2 = jnp.min(jnp.where(pm == p2, idx, NE), axis=-1, keepdims=True)
    return jnp.where((idx == i1) | (idx == i2), p, 0.0) / (p1 + p2)


def _moe_body(x2, refs, qkv_s, o_s, fast_tail):
    (inwt, inb, qkvwt, qkvb, owt, ob, gwt, gb, w1t, b1, w2t, b2,
     vwt, vb, cwt, cb) = refs
    rows = x2.shape[0]
    xi = _dot(x2, inwt[...]) + inb[...]
    qkv = _dot(xi, qkvwt[...]) + qkvb[...]
    qkv_s[...] = qkv.reshape(B, P, 3 * D)

    def attn_step(p_, _):
        xp3 = qkv_s[:, pl.ds(p_, 1), :].reshape(B, 3 * D)
        o = _attn_patch(xp3[:, :D], xp3[:, D:2 * D], xp3[:, 2 * D:])
        o_s[:, pl.ds(p_, 1), :] = o.reshape(B, 1, D)
        return 0

    jax.lax.fori_loop(0, P, attn_step, 0, unroll=4)
    xo = _dot(o_s[...].reshape(rows, D), owt[...]) + ob[...]
    wi = _gate_weights(_dot(xo, gwt[...]) + gb[...])
    edot = _dotf if fast_tail else _dot

    def exp_step(i, acc):
        w1 = w1t[pl.ds(i, 1)].reshape(D, HID)
        h = jnp.maximum(edot(xo, w1) + b1[pl.ds(i, 1)].reshape(1, HID), 0.0)
        w2 = w2t[pl.ds(i, 1)].reshape(HID, D)
        eo = edot(h, w2) + b2[pl.ds(i, 1)].reshape(1, D)
        eidx = jax.lax.broadcasted_iota(jnp.int32, wi.shape, 1)
        wsel = jnp.sum(jnp.where(eidx == i, wi, 0.0), axis=1, keepdims=True)
        return acc + eo * wsel

    acc = jax.lax.fori_loop(0, NE, exp_step,
                            jnp.zeros((rows, D), jnp.float32), unroll=4)
    fv = (edot(acc, vwt[...]) + vb[...])
    cls = _dotf(acc, cwt[...]) + cb[...]
    return fv.reshape(B, P, D), cls.reshape(B, P, D)


def _k1_body(xp_ref, pos_ref, pwt_ref, *rest):
    refs = rest[:16]
    fv_ref, cls_ref, qkv_s, o_s = rest[16:]
    x = xp_ref[...].reshape(B * P, PD)
    x2 = _dot(x, pwt_ref[...])
    x2 = (x2.reshape(B, P, D) + pos_ref[0][None]).reshape(B * P, D)
    fv, cls = _moe_body(x2, refs, qkv_s, o_s, False)
    fv_ref[...] = fv.reshape(B, 1, P, D)
    cls_ref[...] = cls.reshape(B, 1, P, D)


def _k2_body(xin_ref, *rest):
    refs = rest[:16]
    fv_ref, cls_ref, qkv_s, o_s = rest[16:]
    x2 = xin_ref[...].reshape(B * P, D)
    fv, cls = _moe_body(x2, refs, qkv_s, o_s, True)
    fv_ref[...] = fv.reshape(B, 1, P, D)
    cls_ref[...] = cls.reshape(B, 1, P, D)


def _moe_args(mp, vWt, vb, cWt, cb):
    return (
        mp["inW"].T, mp["inb"].reshape(1, D),
        mp["qkvW"].T, mp["qkvb"].reshape(1, 3 * D),
        mp["oW"].T, mp["ob"].reshape(1, D),
        mp["gW"].T, mp["gb"].reshape(1, NE),
        mp["W1"].transpose(0, 2, 1), mp["b1"].reshape(NE, 1, HID),
        mp["W2"].transpose(0, 2, 1), mp["b2"].reshape(NE, 1, D),
        vWt, vb, cWt, cb,
    )


def _moe_specs():
    c2 = lambda shp: pl.BlockSpec(shp, lambda i: (0, 0))
    c3 = lambda shp: pl.BlockSpec(shp, lambda i: (0, 0, 0))
    return [
        c2((D, D)), c2((1, D)),
        c2((D, 3 * D)), c2((1, 3 * D)),
        c2((D, D)), c2((1, D)),
        c2((D, NE)), c2((1, NE)),
        c3((NE, D, HID)), c3((NE, 1, HID)),
        c3((NE, HID, D)), c3((NE, 1, D)),
        c2((D, D)), c2((1, D)), c2((D, D)), c2((1, D)),
    ]


NG = NPATCH // P


def _blk(last):
    return pl.BlockSpec((B, 1, P, last), lambda i: (0, i, 0, 0))


def kernel(x, params):
    n = IMG // PS
    xp4 = (x.reshape(B, n, PS, n, PS)
            .transpose(0, 1, 3, 2, 4)
            .reshape(B, NG, P, PD))
    pos = (params["pos"].reshape(NPATCH, D) +
           params["pb"].reshape(1, D)).reshape(NG, P, D)
    vWt = params["vW"].T
    vb = params["vb"].reshape(1, D)
    cWt = params["cW"].T
    cb = params["cb"].reshape(1, D)
    out_sh = jax.ShapeDtypeStruct((B, NG, P, D), jnp.float32)

    fv1, cls1 = pl.pallas_call(
        _k1_body,
        grid=(NG,),
        in_specs=[_blk(PD),
                  pl.BlockSpec((1, P, D), lambda i: (i, 0, 0)),
                  pl.BlockSpec((PD, D), lambda i: (0, 0))] + _moe_specs(),
        out_specs=[_blk(D), _blk(D)],
        out_shape=[out_sh, out_sh],
        scratch_shapes=[pltpu.VMEM((B, P, 3 * D), jnp.float32),
                        pltpu.VMEM((B, P, D), jnp.float32)],
    )(xp4, pos, params["pW"].T, *_moe_args(params["moe1"], vWt, vb, cWt, cb))

    fv2, cls2 = pl.pallas_call(
        _k2_body,
        grid=(NG,),
        in_specs=[_blk(D)] + _moe_specs(),
        out_specs=[_blk(D), _blk(D)],
        out_shape=[out_sh, out_sh],
        scratch_shapes=[pltpu.VMEM((B, P, 3 * D), jnp.float32),
                        pltpu.VMEM((B, P, D), jnp.float32)],
    )(fv1, *_moe_args(params["moe2"], vWt, vb, cWt, cb))

    sh = (B, NPATCH, D)
    return (fv1.reshape(sh), fv2.reshape(sh),
            cls1.reshape(sh), cls2.reshape(sh))


# speedup vs baseline: 1.6204x; 1.0695x over previous
import functools

import jax
import jax.numpy as jnp
import numpy as np
from jax.experimental import pallas as pl
from jax.experimental.pallas import tpu as pltpu

D = 768
PS = 16
IMG = 224
NPATCH = (IMG // PS) ** 2
PD = PS * PS
NE = 8
NH = 8
DH = D // NH
HID = 256
B = 32
T = B * NPATCH
P = 14
HP = jax.lax.Precision.HIGHEST

SCALE = 1.0 / np.sqrt(DH)


_STD = (((1,), (0,)), ((), ()))
_TR = (((1,), (1,)), ((), ()))


def _dot(a, b):
    return jnp.dot(a, b, preferred_element_type=jnp.float32, precision=HP)


def _split(a):
    ah = a.astype(jnp.bfloat16)
    return ah, (a - ah.astype(jnp.float32)).astype(jnp.bfloat16)


def _dotf(a, b):
    ah, al = _split(a)
    bh, bl = _split(b)
    d = lambda u, v: jax.lax.dot_general(
        u, v, _STD, preferred_element_type=jnp.float32)
    return d(ah, bh) + d(ah, bl) + d(al, bh)


def _attn_patch(q, k, v):
    q8 = jnp.concatenate([q[:, h * DH:(h + 1) * DH] for h in range(NH)], 0)
    k8 = jnp.concatenate([k[:, h * DH:(h + 1) * DH] for h in range(NH)], 0)
    v8 = jnp.concatenate([v[:, h * DH:(h + 1) * DH] for h in range(NH)], 0)
    s = jax.lax.dot_general(q8, k8, _TR, preferred_element_type=jnp.float32,
                            precision=HP) * SCALE
    n = NH * B
    blk = (jax.lax.broadcasted_iota(jnp.int32, (n, n), 0) // B ==
           jax.lax.broadcasted_iota(jnp.int32, (n, n), 1) // B)
    s = jnp.where(blk, s, -1e30)
    m = jnp.max(s, axis=-1, keepdims=True)
    e = jnp.exp(s - m)
    pa = e / jnp.sum(e, axis=-1, keepdims=True)
    o8 = _dot(pa, v8)
    return jnp.concatenate([o8[h * B:(h + 1) * B, :] for h in range(NH)], 1)


def _gate_weights(logits):
    m = jnp.max(logits, axis=-1, keepdims=True)
    e = jnp.exp(logits - m)
    p = e / jnp.sum(e, axis=-1, keepdims=True)
    idx = jax.lax.broadcasted_iota(jnp.int32, p.shape, 1)
    p1 = jnp.max(p, axis=-1, keepdims=True)
    i1 = jnp.min(jnp.where(p == p1, idx, NE), axis=-1, keepdims=True)
    pm = jnp.where(idx == i1, -jnp.inf, p)
    p2 = jnp.max(pm, axis=-1, keepdims=True)
    i2 = jnp.min(jnp.where(pm == p2, idx, NE), axis=-1, keepdims=True)
    return jnp.where((idx == i1) | (idx == i2), p, 0.0) / (p1 + p2)


def _moe_body(x2, refs, qkv_s, o_s, fast_tail):
    (inwt, inb, qkvwt, qkvb, owt, ob, gwt, gb, w1t, b1, w2t, b2,
     vwt, vb, cwt, cb) = refs
    rows = x2.shape[0]
    xi = _dot(x2, inwt[...]) + inb[...]
    qkv = _dot(xi, qkvwt[...]) + qkvb[...]
    qkv_s[...] = qkv.reshape(B, P, 3 * D)

    def attn_step(p_, _):
        xp3 = qkv_s[:, pl.ds(p_, 1), :].reshape(B, 3 * D)
        o = _attn_patch(xp3[:, :D], xp3[:, D:2 * D], xp3[:, 2 * D:])
        o_s[:, pl.ds(p_, 1), :] = o.reshape(B, 1, D)
        return 0

    jax.lax.fori_loop(0, P, attn_step, 0, unroll=7)
    xo = _dot(o_s[...].reshape(rows, D), owt[...]) + ob[...]
    wi = _gate_weights(_dot(xo, gwt[...]) + gb[...])
    edot = _dotf if fast_tail else _dot

    def exp_step(i, acc):
        w1 = w1t[pl.ds(i, 1)].reshape(D, HID)
        h = jnp.maximum(edot(xo, w1) + b1[pl.ds(i, 1)].reshape(1, HID), 0.0)
        w2 = w2t[pl.ds(i, 1)].reshape(HID, D)
        eo = edot(h, w2) + b2[pl.ds(i, 1)].reshape(1, D)
        eidx = jax.lax.broadcasted_iota(jnp.int32, wi.shape, 1)
        wsel = jnp.sum(jnp.where(eidx == i, wi, 0.0), axis=1, keepdims=True)
        return acc + eo * wsel

    acc = jax.lax.fori_loop(0, NE, exp_step,
                            jnp.zeros((rows, D), jnp.float32), unroll=8)
    fv = (edot(acc, vwt[...]) + vb[...])
    cls = _dotf(acc, cwt[...]) + cb[...]
    return fv.reshape(B, P, D), cls.reshape(B, P, D)


def _k1_body(xp_ref, pos_ref, pwt_ref, *rest):
    refs = rest[:16]
    fv_ref, cls_ref, qkv_s, o_s = rest[16:]
    x = xp_ref[...].reshape(B * P, PD)
    x2 = _dot(x, pwt_ref[...])
    x2 = (x2.reshape(B, P, D) + pos_ref[0][None]).reshape(B * P, D)
    fv, cls = _moe_body(x2, refs, qkv_s, o_s, False)
    fv_ref[...] = fv.reshape(B, 1, P, D)
    cls_ref[...] = cls.reshape(B, 1, P, D)


def _k2_body(xin_ref, *rest):
    refs = rest[:16]
    fv_ref, cls_ref, qkv_s, o_s = rest[16:]
    x2 = xin_ref[...].reshape(B * P, D)
    fv, cls = _moe_body(x2, refs, qkv_s, o_s, True)
    fv_ref[...] = fv.reshape(B, 1, P, D)
    cls_ref[...] = cls.reshape(B, 1, P, D)


def _moe_args(mp, vWt, vb, cWt, cb):
    return (
        mp["inW"].T, mp["inb"].reshape(1, D),
        mp["qkvW"].T, mp["qkvb"].reshape(1, 3 * D),
        mp["oW"].T, mp["ob"].reshape(1, D),
        mp["gW"].T, mp["gb"].reshape(1, NE),
        mp["W1"].transpose(0, 2, 1), mp["b1"].reshape(NE, 1, HID),
        mp["W2"].transpose(0, 2, 1), mp["b2"].reshape(NE, 1, D),
        vWt, vb, cWt, cb,
    )


def _moe_specs():
    c2 = lambda shp: pl.BlockSpec(shp, lambda i: (0, 0))
    c3 = lambda shp: pl.BlockSpec(shp, lambda i: (0, 0, 0))
    return [
        c2((D, D)), c2((1, D)),
        c2((D, 3 * D)), c2((1, 3 * D)),
        c2((D, D)), c2((1, D)),
        c2((D, NE)), c2((1, NE)),
        c3((NE, D, HID)), c3((NE, 1, HID)),
        c3((NE, HID, D)), c3((NE, 1, D)),
        c2((D, D)), c2((1, D)), c2((D, D)), c2((1, D)),
    ]


NG = NPATCH // P


def _blk(last):
    return pl.BlockSpec((B, 1, P, last), lambda i: (0, i, 0, 0))


def kernel(x, params):
    n = IMG // PS
    xp4 = (x.reshape(B, n, PS, n, PS)
            .transpose(0, 1, 3, 2, 4)
            .reshape(B, NG, P, PD))
    pos = (params["pos"].reshape(NPATCH, D) +
           params["pb"].reshape(1, D)).reshape(NG, P, D)
    vWt = params["vW"].T
    vb = params["vb"].reshape(1, D)
    cWt = params["cW"].T
    cb = params["cb"].reshape(1, D)
    out_sh = jax.ShapeDtypeStruct((B, NG, P, D), jnp.float32)

    fv1, cls1 = pl.pallas_call(
        _k1_body,
        grid=(NG,),
        in_specs=[_blk(PD),
                  pl.BlockSpec((1, P, D), lambda i: (i, 0, 0)),
                  pl.BlockSpec((PD, D), lambda i: (0, 0))] + _moe_specs(),
        out_specs=[_blk(D), _blk(D)],
        out_shape=[out_sh, out_sh],
        scratch_shapes=[pltpu.VMEM((B, P, 3 * D), jnp.float32),
                        pltpu.VMEM((B, P, D), jnp.float32)],
    )(xp4, pos, params["pW"].T, *_moe_args(params["moe1"], vWt, vb, cWt, cb))

    fv2, cls2 = pl.pallas_call(
        _k2_body,
        grid=(NG,),
        in_specs=[_blk(D)] + _moe_specs(),
        out_specs=[_blk(D), _blk(D)],
        out_shape=[out_sh, out_sh],
        scratch_shapes=[pltpu.VMEM((B, P, 3 * D), jnp.float32),
                        pltpu.VMEM((B, P, D), jnp.float32)],
    )(fv1, *_moe_args(params["moe2"], vWt, vb, cWt, cb))

    sh = (B, NPATCH, D)
    return (fv1.reshape(sh), fv2.reshape(sh),
            cls1.reshape(sh), cls2.reshape(sh))


# speedup vs baseline: 1.7666x; 1.0902x over previous
import functools

import jax
import jax.numpy as jnp
import numpy as np
from jax.experimental import pallas as pl
from jax.experimental.pallas import tpu as pltpu

D = 768
PS = 16
IMG = 224
NPATCH = (IMG // PS) ** 2
PD = PS * PS
NE = 8
NH = 8
DH = D // NH
HID = 256
B = 32
T = B * NPATCH
P = 14
HP = jax.lax.Precision.HIGHEST

SCALE = 1.0 / np.sqrt(DH)


_STD = (((1,), (0,)), ((), ()))
_TR = (((1,), (1,)), ((), ()))


def _dot(a, b):
    bh = b.astype(jnp.bfloat16)
    r1 = b - bh.astype(jnp.float32)
    bm = r1.astype(jnp.bfloat16)
    bl = (r1 - bm.astype(jnp.float32)).astype(jnp.bfloat16)
    d = lambda v: jax.lax.dot_general(
        a, v, _STD, preferred_element_type=jnp.float32)
    return d(bh) + d(bm) + d(bl)


def _split(a):
    ah = a.astype(jnp.bfloat16)
    return ah, (a - ah.astype(jnp.float32)).astype(jnp.bfloat16)


def _dotf(a, b):
    ah, al = _split(a)
    bh, bl = _split(b)
    d = lambda u, v: jax.lax.dot_general(
        u, v, _STD, preferred_element_type=jnp.float32)
    return d(ah, bh) + d(ah, bl) + d(al, bh)


def _attn_patch(q, k, v):
    q8 = jnp.concatenate([q[:, h * DH:(h + 1) * DH] for h in range(NH)], 0)
    k8 = jnp.concatenate([k[:, h * DH:(h + 1) * DH] for h in range(NH)], 0)
    v8 = jnp.concatenate([v[:, h * DH:(h + 1) * DH] for h in range(NH)], 0)
    s = jax.lax.dot_general(q8, k8, _TR, preferred_element_type=jnp.float32,
                            precision=HP) * SCALE
    n = NH * B
    blk = (jax.lax.broadcasted_iota(jnp.int32, (n, n), 0) // B ==
           jax.lax.broadcasted_iota(jnp.int32, (n, n), 1) // B)
    s = jnp.where(blk, s, -1e30)
    m = jnp.max(s, axis=-1, keepdims=True)
    e = jnp.exp(s - m)
    pa = e / jnp.sum(e, axis=-1, keepdims=True)
    o8 = _dot(pa, v8)
    return jnp.concatenate([o8[h * B:(h + 1) * B, :] for h in range(NH)], 1)


def _gate_weights(logits):
    m = jnp.max(logits, axis=-1, keepdims=True)
    e = jnp.exp(logits - m)
    p = e / jnp.sum(e, axis=-1, keepdims=True)
    idx = jax.lax.broadcasted_iota(jnp.int32, p.shape, 1)
    p1 = jnp.max(p, axis=-1, keepdims=True)
    i1 = jnp.min(jnp.where(p == p1, idx, NE), axis=-1, keepdims=True)
    pm = jnp.where(idx == i1, -jnp.inf, p)
    p2 = jnp.max(pm, axis=-1, keepdims=True)
    i2 = jnp.min(jnp.where(pm == p2, idx, NE), axis=-1, keepdims=True)
    return jnp.where((idx == i1) | (idx == i2), p, 0.0) / (p1 + p2)


def _moe_body(x2, refs, qkv_s, fast_tail):
    (inwt, inb, qkvwt, qkvb, owt, ob, gwt, gb, w1t, b1, w2t, b2,
     vwt, vb, cwt, cb) = refs
    rows = x2.shape[0]
    xi = _dot(x2, inwt[...]) + inb[...]
    qkv = jnp.dot(xi, qkvwt[...], preferred_element_type=jnp.float32,
                  precision=HP) + qkvb[...]
    qkv_s[...] = qkv.reshape(B, P, 3 * D)

    def attn_step(p_, _):
        xp3 = qkv_s[:, pl.ds(p_, 1), :].reshape(B, 3 * D)
        o = _attn_patch(xp3[:, :D], xp3[:, D:2 * D], xp3[:, 2 * D:])
        qkv_s[:, pl.ds(p_, 1), 0:D] = o.reshape(B, 1, D)
        return 0

    jax.lax.fori_loop(0, P, attn_step, 0, unroll=7)
    xo = _dot(qkv_s[:, :, 0:D].reshape(rows, D), owt[...]) + ob[...]
    wi = _gate_weights(_dot(xo, gwt[...]) + gb[...])
    edot = _dotf if fast_tail else _dot

    def exp_step(i, acc):
        w1 = w1t[pl.ds(i, 1)].reshape(D, HID)
        h = jnp.maximum(edot(xo, w1) + b1[pl.ds(i, 1)].reshape(1, HID), 0.0)
        w2 = w2t[pl.ds(i, 1)].reshape(HID, D)
        eo = edot(h, w2) + b2[pl.ds(i, 1)].reshape(1, D)
        eidx = jax.lax.broadcasted_iota(jnp.int32, wi.shape, 1)
        wsel = jnp.sum(jnp.where(eidx == i, wi, 0.0), axis=1, keepdims=True)
        return acc + eo * wsel

    acc = jax.lax.fori_loop(0, NE, exp_step,
                            jnp.zeros((rows, D), jnp.float32), unroll=8)
    fv = (edot(acc, vwt[...]) + vb[...])
    cls = _dotf(acc, cwt[...]) + cb[...]
    return fv.reshape(B, P, D), cls.reshape(B, P, D)


def _k1_body(xp_ref, pos_ref, pwt_ref, *rest):
    refs = rest[:16]
    fv_ref, cls_ref, qkv_s = rest[16:]
    x = xp_ref[...].reshape(B * P, PD)
    x2 = _dot(x, pwt_ref[...])
    x2 = (x2.reshape(B, P, D) + pos_ref[0][None]).reshape(B * P, D)
    fv, cls = _moe_body(x2, refs, qkv_s, False)
    fv_ref[...] = fv.reshape(B, 1, P, D)
    cls_ref[...] = cls.reshape(B, 1, P, D)


def _k2_body(xin_ref, *rest):
    refs = rest[:16]
    fv_ref, cls_ref, qkv_s = rest[16:]
    x2 = xin_ref[...].reshape(B * P, D)
    fv, cls = _moe_body(x2, refs, qkv_s, True)
    fv_ref[...] = fv.reshape(B, 1, P, D)
    cls_ref[...] = cls.reshape(B, 1, P, D)


def _moe_args(mp, vWt, vb, cWt, cb):
    return (
        mp["inW"].T, mp["inb"].reshape(1, D),
        mp["qkvW"].T, mp["qkvb"].reshape(1, 3 * D),
        mp["oW"].T, mp["ob"].reshape(1, D),
        mp["gW"].T, mp["gb"].reshape(1, NE),
        mp["W1"].transpose(0, 2, 1), mp["b1"].reshape(NE, 1, HID),
        mp["W2"].transpose(0, 2, 1), mp["b2"].reshape(NE, 1, D),
        vWt, vb, cWt, cb,
    )


def _moe_specs():
    c2 = lambda shp: pl.BlockSpec(shp, lambda i: (0, 0))
    c3 = lambda shp: pl.BlockSpec(shp, lambda i: (0, 0, 0))
    return [
        c2((D, D)), c2((1, D)),
        c2((D, 3 * D)), c2((1, 3 * D)),
        c2((D, D)), c2((1, D)),
        c2((D, NE)), c2((1, NE)),
        c3((NE, D, HID)), c3((NE, 1, HID)),
        c3((NE, HID, D)), c3((NE, 1, D)),
        c2((D, D)), c2((1, D)), c2((D, D)), c2((1, D)),
    ]


NG = NPATCH // P


def _blk(last):
    return pl.BlockSpec((B, 1, P, last), lambda i: (0, i, 0, 0))


def kernel(x, params):
    n = IMG // PS
    xp4 = (x.reshape(B, n, PS, n, PS)
            .transpose(0, 1, 3, 2, 4)
            .reshape(B, NG, P, PD))
    pos = (params["pos"].reshape(NPATCH, D) +
           params["pb"].reshape(1, D)).reshape(NG, P, D)
    vWt = params["vW"].T
    vb = params["vb"].reshape(1, D)
    cWt = params["cW"].T
    cb = params["cb"].reshape(1, D)
    out_sh = jax.ShapeDtypeStruct((B, NG, P, D), jnp.float32)

    fv1, cls1 = pl.pallas_call(
        _k1_body,
        grid=(NG,),
        in_specs=[_blk(PD),
                  pl.BlockSpec((1, P, D), lambda i: (i, 0, 0)),
                  pl.BlockSpec((PD, D), lambda i: (0, 0))] + _moe_specs(),
        out_specs=[_blk(D), _blk(D)],
        out_shape=[out_sh, out_sh],
        scratch_shapes=[pltpu.VMEM((B, P, 3 * D), jnp.float32)],
    )(xp4, pos, params["pW"].T, *_moe_args(params["moe1"], vWt, vb, cWt, cb))

    fv2, cls2 = pl.pallas_call(
        _k2_body,
        grid=(NG,),
        in_specs=[_blk(D)] + _moe_specs(),
        out_specs=[_blk(D), _blk(D)],
        out_shape=[out_sh, out_sh],
        scratch_shapes=[pltpu.VMEM((B, P, 3 * D), jnp.float32)],
    )(fv1, *_moe_args(params["moe2"], vWt, vb, cWt, cb))

    sh = (B, NPATCH, D)
    return (fv1.reshape(sh), fv2.reshape(sh),
            cls1.reshape(sh), cls2.reshape(sh))


# speedup vs baseline: 1.9704x; 1.1154x over previous
import functools

import jax
import jax.numpy as jnp
import numpy as np
from jax.experimental import pallas as pl
from jax.experimental.pallas import tpu as pltpu

D = 768
PS = 16
IMG = 224
NPATCH = (IMG // PS) ** 2
PD = PS * PS
NE = 8
NH = 8
DH = D // NH
HID = 256
B = 32
T = B * NPATCH
P = 14
HP = jax.lax.Precision.HIGHEST

SCALE = 1.0 / np.sqrt(DH)


_STD = (((1,), (0,)), ((), ()))
_TR = (((1,), (1,)), ((), ()))


def _dot(a, b):
    bh = b.astype(jnp.bfloat16)
    r1 = b - bh.astype(jnp.float32)
    bm = r1.astype(jnp.bfloat16)
    bl = (r1 - bm.astype(jnp.float32)).astype(jnp.bfloat16)
    d = lambda v: jax.lax.dot_general(
        a, v, _STD, preferred_element_type=jnp.float32)
    return d(bh) + d(bm) + d(bl)


def _split(a):
    ah = a.astype(jnp.bfloat16)
    return ah, (a - ah.astype(jnp.float32)).astype(jnp.bfloat16)


def _dotf(a, b):
    ah, al = _split(a)
    bh, bl = _split(b)
    d = lambda u, v: jax.lax.dot_general(
        u, v, _STD, preferred_element_type=jnp.float32)
    return d(ah, bh) + d(ah, bl) + d(al, bh)


def _attn_patch(q, k, v):
    q8 = jnp.concatenate([q[:, h * DH:(h + 1) * DH] for h in range(NH)], 0)
    k8 = jnp.concatenate([k[:, h * DH:(h + 1) * DH] for h in range(NH)], 0)
    v8 = jnp.concatenate([v[:, h * DH:(h + 1) * DH] for h in range(NH)], 0)
    s = _mix3(q8, k8, _TR) * SCALE
    n = NH * B
    blk = (jax.lax.broadcasted_iota(jnp.int32, (n, n), 0) // B ==
           jax.lax.broadcasted_iota(jnp.int32, (n, n), 1) // B)
    s = jnp.where(blk, s, -1e30)
    m = jnp.max(s, axis=-1, keepdims=True)
    e = jnp.exp(s - m)
    pa = e / jnp.sum(e, axis=-1, keepdims=True)
    o8 = _mix3(pa, v8, _STD)
    return jnp.concatenate([o8[h * B:(h + 1) * B, :] for h in range(NH)], 1)


def _mix3(a, b, dims):
    bh = b.astype(jnp.bfloat16)
    r1 = b - bh.astype(jnp.float32)
    bm = r1.astype(jnp.bfloat16)
    bl = (r1 - bm.astype(jnp.float32)).astype(jnp.bfloat16)
    d = lambda v: jax.lax.dot_general(
        a, v, dims, preferred_element_type=jnp.float32)
    return d(bh) + d(bm) + d(bl)


def _gate_weights(logits):
    m = jnp.max(logits, axis=-1, keepdims=True)
    e = jnp.exp(logits - m)
    p = e / jnp.sum(e, axis=-1, keepdims=True)
    idx = jax.lax.broadcasted_iota(jnp.int32, p.shape, 1)
    p1 = jnp.max(p, axis=-1, keepdims=True)
    i1 = jnp.min(jnp.where(p == p1, idx, NE), axis=-1, keepdims=True)
    pm = jnp.where(idx == i1, -jnp.inf, p)
    p2 = jnp.max(pm, axis=-1, keepdims=True)
    i2 = jnp.min(jnp.where(pm == p2, idx, NE), axis=-1, keepdims=True)
    return jnp.where((idx == i1) | (idx == i2), p, 0.0) / (p1 + p2)


def _moe_body(x2, refs, qkv_s, fast_tail):
    (inwt, inb, qwh, qwm, qwl, qkvb, owt, ob, gwt, gb, w1t, b1, w2t, b2,
     vwt, vb, cwt, cb) = refs
    rows = x2.shape[0]
    xi = _dot(x2, inwt[...]) + inb[...]
    dq = lambda v: jax.lax.dot_general(
        xi, v, _STD, preferred_element_type=jnp.float32)
    qkv = dq(qwh[...]) + dq(qwm[...]) + dq(qwl[...]) + qkvb[...]
    qkv_s[...] = qkv.reshape(B, P, 3 * D)

    def attn_step(p_, _):
        xp3 = qkv_s[:, pl.ds(p_, 1), :].reshape(B, 3 * D)
        o = _attn_patch(xp3[:, :D], xp3[:, D:2 * D], xp3[:, 2 * D:])
        qkv_s[:, pl.ds(p_, 1), 0:D] = o.reshape(B, 1, D)
        return 0

    jax.lax.fori_loop(0, P, attn_step, 0, unroll=7)
    xo = _dot(qkv_s[:, :, 0:D].reshape(rows, D), owt[...]) + ob[...]
    wi = _gate_weights(_dot(xo, gwt[...]) + gb[...])
    edot = _dotf if fast_tail else _dot

    def exp_step(i, acc):
        w1 = w1t[pl.ds(i, 1)].reshape(D, HID)
        h = jnp.maximum(edot(xo, w1) + b1[pl.ds(i, 1)].reshape(1, HID), 0.0)
        w2 = w2t[pl.ds(i, 1)].reshape(HID, D)
        eo = edot(h, w2) + b2[pl.ds(i, 1)].reshape(1, D)
        eidx = jax.lax.broadcasted_iota(jnp.int32, wi.shape, 1)
        wsel = jnp.sum(jnp.where(eidx == i, wi, 0.0), axis=1, keepdims=True)
        return acc + eo * wsel

    acc = jax.lax.fori_loop(0, NE, exp_step,
                            jnp.zeros((rows, D), jnp.float32), unroll=8)
    fv = (edot(acc, vwt[...]) + vb[...])
    cls = _dotf(acc, cwt[...]) + cb[...]
    return fv.reshape(B, P, D), cls.reshape(B, P, D)


def _k1_body(xp_ref, pos_ref, pwt_ref, *rest):
    refs = rest[:18]
    fv_ref, cls_ref, qkv_s = rest[18:]
    x = xp_ref[...].reshape(B * P, PD)
    x2 = _dot(x, pwt_ref[...])
    x2 = (x2.reshape(B, P, D) + pos_ref[0][None]).reshape(B * P, D)
    fv, cls = _moe_body(x2, refs, qkv_s, False)
    fv_ref[...] = fv.reshape(B, 1, P, D)
    cls_ref[...] = cls.reshape(B, 1, P, D)


def _k2_body(xin_ref, *rest):
    refs = rest[:18]
    fv_ref, cls_ref, qkv_s = rest[18:]
    x2 = xin_ref[...].reshape(B * P, D)
    fv, cls = _moe_body(x2, refs, qkv_s, True)
    fv_ref[...] = fv.reshape(B, 1, P, D)
    cls_ref[...] = cls.reshape(B, 1, P, D)


def _split3(w):
    wh = w.astype(jnp.bfloat16)
    r1 = w - wh.astype(jnp.float32)
    wm = r1.astype(jnp.bfloat16)
    wl = (r1 - wm.astype(jnp.float32)).astype(jnp.bfloat16)
    return wh, wm, wl


def _moe_args(mp, vWt, vb, cWt, cb):
    qwh, qwm, qwl = _split3(mp["qkvW"].T)
    return (
        mp["inW"].T, mp["inb"].reshape(1, D),
        qwh, qwm, qwl, mp["qkvb"].reshape(1, 3 * D),
        mp["oW"].T, mp["ob"].reshape(1, D),
        mp["gW"].T, mp["gb"].reshape(1, NE),
        mp["W1"].transpose(0, 2, 1), mp["b1"].reshape(NE, 1, HID),
        mp["W2"].transpose(0, 2, 1), mp["b2"].reshape(NE, 1, D),
        vWt, vb, cWt, cb,
    )


def _moe_specs():
    c2 = lambda shp: pl.BlockSpec(shp, lambda i: (0, 0))
    c3 = lambda shp: pl.BlockSpec(shp, lambda i: (0, 0, 0))
    return [
        c2((D, D)), c2((1, D)),
        c2((D, 3 * D)), c2((D, 3 * D)), c2((D, 3 * D)), c2((1, 3 * D)),
        c2((D, D)), c2((1, D)),
        c2((D, NE)), c2((1, NE)),
        c3((NE, D, HID)), c3((NE, 1, HID)),
        c3((NE, HID, D)), c3((NE, 1, D)),
        c2((D, D)), c2((1, D)), c2((D, D)), c2((1, D)),
    ]


NG = NPATCH // P


def _blk(last):
    return pl.BlockSpec((B, 1, P, last), lambda i: (0, i, 0, 0))


def kernel(x, params):
    n = IMG // PS
    xp4 = (x.reshape(B, n, PS, n, PS)
            .transpose(0, 1, 3, 2, 4)
            .reshape(B, NG, P, PD))
    pos = (params["pos"].reshape(NPATCH, D) +
           params["pb"].reshape(1, D)).reshape(NG, P, D)
    vWt = params["vW"].T
    vb = params["vb"].reshape(1, D)
    cWt = params["cW"].T
    cb = params["cb"].reshape(1, D)
    out_sh = jax.ShapeDtypeStruct((B, NG, P, D), jnp.float32)

    fv1, cls1 = pl.pallas_call(
        _k1_body,
        grid=(NG,),
        in_specs=[_blk(PD),
                  pl.BlockSpec((1, P, D), lambda i: (i, 0, 0)),
                  pl.BlockSpec((PD, D), lambda i: (0, 0))] + _moe_specs(),
        out_specs=[_blk(D), _blk(D)],
        out_shape=[out_sh, out_sh],
        scratch_shapes=[pltpu.VMEM((B, P, 3 * D), jnp.float32)],
    )(xp4, pos, params["pW"].T, *_moe_args(params["moe1"], vWt, vb, cWt, cb))

    fv2, cls2 = pl.pallas_call(
        _k2_body,
        grid=(NG,),
        in_specs=[_blk(D)] + _moe_specs(),
        out_specs=[_blk(D), _blk(D)],
        out_shape=[out_sh, out_sh],
        scratch_shapes=[pltpu.VMEM((B, P, 3 * D), jnp.float32)],
    )(fv1, *_moe_args(params["moe2"], vWt, vb, cWt, cb))

    sh = (B, NPATCH, D)
    return (fv1.reshape(sh), fv2.reshape(sh),
            cls1.reshape(sh), cls2.reshape(sh))
